# baseline (speedup 1.0000x reference)
import functools

import jax
import jax.numpy as jnp
from jax import lax
from jax.experimental import pallas as pl
from jax.experimental.pallas import tpu as pltpu
from jax.experimental.pallas import tpu_sc as plsc

B, T, V, D = 16384, 12, 28, 24
LANES = 16

NUM_CORES = 2
NUM_SUBCORES = 16
NW = NUM_CORES * NUM_SUBCORES
BPW = B // NW
BT = 128
UNITS = BPW // BT
DT = D // 8
VP = D + 1
TP = V * VP
NREP = 4
REP = T * TP + 4


def _sc_body(x_hbm, word_hbm, pos_hbm, out_hbm, xch_v, wt_v, pos_v, ft_v, blk_v, sem):
    wid = lax.axis_index("s") * NUM_CORES + lax.axis_index("c")
    b0 = wid * BPW

    pltpu.sync_copy(word_hbm, wt_v.at[pl.ds(0, V * D)])
    pltpu.sync_copy(pos_hbm, pos_v.at[pl.ds(0, T * D)])
    pltpu.sync_copy(x_hbm.at[:, pl.ds(b0, BPW)], xch_v)

    lane = lax.iota(jnp.int32, LANES)
    repsel = lax.shift_right_logical(lane, 2) * REP

    def ft_t(t, _):
        def ft_grp(m, _):
            jr = m * LANES + lane
            q = lax.shift_right_logical(jr * 10486, 18)
            wv = plsc.load_gather(wt_v, [jr - q])
            pvv = plsc.load_gather(pos_v, [jr - q * VP + t * D])
            fv = wv + pvv
            for r in range(NREP):
                ft_v[pl.ds(r * REP + t * TP + m * LANES, LANES)] = fv
            return 0

        lax.fori_loop(0, (TP + LANES - 1) // LANES, ft_grp, 0)
        return 0

    lax.fori_loop(0, T, ft_t, 0)

    copies = []
    for u in range(UNITS):
        bh = wid * UNITS + u
        buf = u % 2
        if u >= 2:
            copies[u - 2].wait()

        def t_loop(t, _):
            for g in range(BT // LANES):
                xv = xch_v[t, pl.ds(u * BT + g * LANES, LANES)]
                xvt = xv * VP + (t * TP + repsel)
                for d in range(D):
                    val = plsc.load_gather(ft_v, [xvt + d])
                    blk_v[buf, t, d // 8, d % 8, pl.ds(g * LANES, LANES)] = val
            return 0

        lax.fori_loop(0, T, t_loop, 0)
        copies.append(pltpu.async_copy(blk_v.at[buf], out_hbm.at[:, :, bh], sem))
    for c in copies[-2:]:
        c.wait()


@jax.jit
def kernel(X, word_table, pos_table):
    x_t = X.T
    wt_flat = word_table.reshape(V * D)
    pos_flat = pos_table.reshape(T * D)

    mesh = plsc.VectorSubcoreMesh(core_axis_name="c", subcore_axis_name="s")
    sc = pl.kernel(
        _sc_body,
        out_type=jax.ShapeDtypeStruct((T, DT, B // BT, 8, BT), jnp.float32),
        mesh=mesh,
        scratch_types=[
            pltpu.VMEM((T, BPW), jnp.int32),
            pltpu.VMEM((V * D + 32,), jnp.float32),
            pltpu.VMEM((T * D + 16,), jnp.float32),
            pltpu.VMEM((NREP * REP + 16,), jnp.float32),
            pltpu.VMEM((2, T, DT, 8, BT), jnp.float32),
            pltpu.SemaphoreType.DMA,
        ],
        compiler_params=pltpu.CompilerParams(
            use_tc_tiling_on_sc=False, needs_layout_passes=False
        ),
    )
    out5 = sc(x_t, wt_flat, pos_flat)
    return jnp.transpose(out5, (2, 4, 0, 1, 3)).reshape(B, T, D)
